# baseline (speedup 1.0000x reference)
import jax
import jax.numpy as jnp
from jax.experimental import pallas as pl
from jax.experimental.pallas import tpu as pltpu

N = 4096
G = 4
D = 256
THR = -1.0
BM = 512


def _edge_kernel(
    x_ref,
    w_ref,
    b_ref,
    seg_ref,
    score_ref,
    adjp_ref,
    bne_ref,
    h_scr,
    sq_scr,
    q_scr,
    ends_scr,
):
    t = pl.program_id(0)

    @pl.when(t == 0)
    def _():
        x = x_ref[...]
        h = jnp.dot(x, w_ref[...], preferred_element_type=jnp.float32) + b_ref[...]
        h_scr[...] = h
        ones = jnp.ones((1, D), jnp.float32)
        sq_scr[...] = jax.lax.dot_general(
            ones, h * h, (((1,), (1,)), ((), ())), preferred_element_type=jnp.float32
        )
        seg_full = seg_ref[...]
        e = jnp.int32(0)
        for k in range(G):
            e = e + jnp.sum((seg_full == k).astype(jnp.int32))
            ends_scr[k] = e
        km = jax.lax.broadcasted_iota(jnp.int32, (BM // 8, BM), 1)
        mm = jax.lax.broadcasted_iota(jnp.int32, (BM // 8, BM), 0)
        pw = (1 << (km & 7)).astype(jnp.float32)
        q_scr[...] = jnp.where((km >> 3) == mm, pw, 0.0)
        bne_ref[...] = jnp.zeros((1, 1, 128), jnp.int32)

    @pl.when(t > 0)
    def _():
        i = t - 1
        hi = h_scr[pl.ds(i * BM, BM), :]
        hfull = h_scr[...]
        dot = jax.lax.dot_general(
            hi + hi, hfull, (((1,), (1,)), ((), ())), preferred_element_type=jnp.float32
        )
        sqi = jnp.sum(hi * hi, axis=1, keepdims=True)
        score = dot - sqi - sq_scr[...]
        score_ref[...] = score

        ends = [ends_scr[k] for k in range(G)]
        row = i * BM + jax.lax.broadcasted_iota(jnp.int32, (BM, 1), 0)
        col = jax.lax.broadcasted_iota(jnp.int32, (1, N), 1)
        segr = sum((row >= ends[k]).astype(jnp.int32) for k in range(G))
        segc = sum((col >= ends[k]).astype(jnp.int32) for k in range(G))
        adjf = jnp.where(
            (score > THR) & (segr == segc) & (row != col), 1.0, 0.0
        ).astype(jnp.float32)
        packed = jax.lax.dot_general(
            q_scr[...], adjf, (((1,), (0,)), ((), ())),
            preferred_element_type=jnp.float32,
        )
        adjp_ref[...] = packed.astype(jnp.uint8)

        rowdeg = jax.lax.dot_general(
            adjf, jnp.ones((1, N), jnp.float32), (((1,), (1,)), ((), ())),
            preferred_element_type=jnp.float32,
        )
        lanes = jax.lax.broadcasted_iota(jnp.int32, (1, 128), 1)
        contrib = jnp.sum(jnp.where(segr == lanes, rowdeg, 0.0), axis=0, keepdims=True)
        bne_ref[...] += contrib.astype(jnp.int32).reshape(1, 1, 128)


def kernel(t, dynamicVariable, segment_ids, W, b):
    x = dynamicVariable
    b2 = b.reshape(1, D)
    seg2d = segment_ids.reshape(1, N).astype(jnp.int32)

    nb = N // BM
    score, adjp, bne3 = pl.pallas_call(
        _edge_kernel,
        grid=(nb + 1,),
        in_specs=[
            pl.BlockSpec((N, D), lambda t: (0, 0)),
            pl.BlockSpec((D, D), lambda t: (0, 0)),
            pl.BlockSpec((1, D), lambda t: (0, 0)),
            pl.BlockSpec((1, N), lambda t: (0, 0)),
        ],
        out_specs=[
            pl.BlockSpec((BM, N), lambda t: (jnp.maximum(t - 1, 0), 0)),
            pl.BlockSpec((BM // 8, N), lambda t: (jnp.maximum(t - 1, 0), 0)),
            pl.BlockSpec((1, 1, 128), lambda t: (0, 0, 0)),
        ],
        out_shape=[
            jax.ShapeDtypeStruct((N, N), jnp.float32),
            jax.ShapeDtypeStruct((N // 8, N), jnp.uint8),
            jax.ShapeDtypeStruct((1, 1, 128), jnp.int32),
        ],
        scratch_shapes=[
            pltpu.VMEM((N, D), jnp.float32),
            pltpu.VMEM((1, N), jnp.float32),
            pltpu.VMEM((BM // 8, BM), jnp.float32),
            pltpu.SMEM((G,), jnp.int32),
        ],
    )(x, W, b2, seg2d)

    bits = jnp.arange(8, dtype=jnp.uint8).reshape(1, 8, 1)
    adj = ((adjp[:, None, :] >> bits) & 1).astype(jnp.bool_).reshape(N, N)
    bne = bne3.reshape(128)[:G]
    return (score, adj, bne)


# speedup vs baseline: 1.2808x; 1.2808x over previous
import jax
import jax.numpy as jnp
from jax.experimental import pallas as pl
from jax.experimental.pallas import tpu as pltpu

N = 4096
G = 4
D = 256
THR = -1.0
BM = 1024


def _edge_kernel(
    x_ref, w_ref, b_ref, seg_ref, score_ref, adj_ref, bne_ref, h_scr, sq_scr, ends_scr
):
    t = pl.program_id(0)

    @pl.when(t == 0)
    def _():
        x = x_ref[...]
        h = jnp.dot(x, w_ref[...], preferred_element_type=jnp.float32) + b_ref[...]
        h_scr[...] = h
        ones = jnp.ones((1, D), jnp.float32)
        sq_scr[...] = jax.lax.dot_general(
            ones, h * h, (((1,), (1,)), ((), ())), preferred_element_type=jnp.float32
        )
        seg_full = seg_ref[...]
        e = jnp.int32(0)
        for k in range(G):
            e = e + jnp.sum((seg_full == k).astype(jnp.int32))
            ends_scr[k] = e
        bne_ref[...] = jnp.zeros((1, 1, 128), jnp.int32)

    @pl.when(t > 0)
    def _():
        i = t - 1
        hi = h_scr[pl.ds(i * BM, BM), :]
        hfull = h_scr[...]
        dot = jax.lax.dot_general(
            hi + hi, hfull, (((1,), (1,)), ((), ())), preferred_element_type=jnp.float32
        )
        sqi = jnp.sum(hi * hi, axis=1, keepdims=True)
        score = dot - sqi - sq_scr[...]
        score_ref[...] = score

        ends = [ends_scr[k] for k in range(G)]
        row = i * BM + jax.lax.broadcasted_iota(jnp.int32, (BM, 1), 0)
        col = jax.lax.broadcasted_iota(jnp.int32, (1, N), 1)
        segr = sum((row >= ends[k]).astype(jnp.int32) for k in range(G))
        segc = sum((col >= ends[k]).astype(jnp.int32) for k in range(G))
        adj = (score > THR) & (segr == segc) & (row != col)
        adj_ref[...] = adj.astype(jnp.int8)

        rowdeg = jnp.sum(adj.astype(jnp.int32), axis=1, keepdims=True)
        lanes = jax.lax.broadcasted_iota(jnp.int32, (1, 128), 1)
        contrib = jnp.sum(jnp.where(segr == lanes, rowdeg, 0), axis=0, keepdims=True)
        bne_ref[...] += contrib.reshape(1, 1, 128)


def kernel(t, dynamicVariable, segment_ids, W, b):
    x = dynamicVariable
    b2 = b.reshape(1, D)
    seg2d = segment_ids.reshape(1, N).astype(jnp.int32)

    nb = N // BM
    score, adj, bne3 = pl.pallas_call(
        _edge_kernel,
        grid=(nb + 1,),
        in_specs=[
            pl.BlockSpec((N, D), lambda t: (0, 0)),
            pl.BlockSpec((D, D), lambda t: (0, 0)),
            pl.BlockSpec((1, D), lambda t: (0, 0)),
            pl.BlockSpec((1, N), lambda t: (0, 0)),
        ],
        out_specs=[
            pl.BlockSpec((BM, N), lambda t: (jnp.maximum(t - 1, 0), 0)),
            pl.BlockSpec((BM, N), lambda t: (jnp.maximum(t - 1, 0), 0)),
            pl.BlockSpec((1, 1, 128), lambda t: (0, 0, 0)),
        ],
        out_shape=[
            jax.ShapeDtypeStruct((N, N), jnp.float32),
            jax.ShapeDtypeStruct((N, N), jnp.int8),
            jax.ShapeDtypeStruct((1, 1, 128), jnp.int32),
        ],
        scratch_shapes=[
            pltpu.VMEM((N, D), jnp.float32),
            pltpu.VMEM((1, N), jnp.float32),
            pltpu.SMEM((G,), jnp.int32),
        ],
    )(x, W, b2, seg2d)

    bne = bne3.reshape(128)[:G]
    return (score, adj.astype(jnp.bool_), bne)
